# baseline (device time: 15878 ns/iter reference)
import jax
import jax.numpy as jnp
from jax import lax
from jax.experimental import pallas as pl
from jax.experimental.pallas import tpu as pltpu

N_DEV = 4
B, SQ, HQ, DH = 2, 256, 4, 64
SKV_SHARD = 256
WINDOW = 128
HD = HQ * DH
D_MODEL = 512
QR = SQ // 4
NEG = -1e9
BF = jnp.bfloat16


def kernel(x, Wq, K_ext, V_ext, Wo):
    K2 = K_ext.reshape(B, SKV_SHARD, HD)
    V2 = V_ext.reshape(B, SKV_SHARD, HD)

    def body(x_ref, wq_ref, k_ref, v_ref, wo_ref, out_ref,
             loc_ctx, loc_stat, rctx_a, rstat_a, rctx_b, rstat_b, gctx,
             sc_send, sc_recv, st_send, st_recv, ag_send, ag_recv):
        my = lax.axis_index("i")
        right = lax.rem(my + 1, N_DEV)
        left = lax.rem(my + N_DEV - 1, N_DEV)
        diag = lax.rem(my + 2, N_DEV)

        bsem = pltpu.get_barrier_semaphore()
        for nbr in (left, right, diag):
            pl.semaphore_signal(bsem, inc=1, device_id=(nbr,),
                                device_id_type=pl.DeviceIdType.MESH)
        pl.semaphore_wait(bsem, N_DEV - 1)

        kv_off = my * SKV_SHARD

        def compute_partial(row0, row1, col0, col1, normalize):
            rows, cols = row1 - row0, col1 - col0
            qi = lax.broadcasted_iota(jnp.int32, (rows, cols), 0) + row0
            kj = (lax.broadcasted_iota(jnp.int32, (rows, cols), 1)
                  + col0 + kv_off)
            msk = jnp.abs(qi - kj) <= WINDOW
            for b in range(B):
                q_b = jnp.dot(x_ref[b, row0:row1, :].astype(BF),
                              wq_ref[:, :].astype(BF),
                              preferred_element_type=jnp.float32)
                cs, ms, ls = [], [], []
                for h in range(HQ):
                    qh = q_b[:, h * DH:(h + 1) * DH].astype(BF)
                    kh = k_ref[b, col0:col1, h * DH:(h + 1) * DH].astype(BF)
                    vh = v_ref[b, col0:col1, h * DH:(h + 1) * DH].astype(BF)
                    s = lax.dot_general(
                        qh, kh, (((1,), (1,)), ((), ())),
                        preferred_element_type=jnp.float32) * 0.125
                    s = jnp.where(msk, s, NEG)
                    m = jnp.max(s, axis=1, keepdims=True)
                    w = jnp.exp(s - m)
                    l = jnp.sum(w, axis=1, keepdims=True)
                    c = jnp.dot(w.astype(BF), vh,
                                preferred_element_type=jnp.float32)
                    cs.append(c / l if normalize else c)
                    ms.append(m)
                    ls.append(l)
                if normalize:
                    gctx[b, row0:row1, :] = jnp.concatenate(
                        cs, axis=1).astype(BF)
                else:
                    loc_ctx[b, row0:row1, :] = jnp.concatenate(
                        cs, axis=1).astype(BF)
                    loc_stat[b, row0:row1, :] = jnp.concatenate(
                        ms + ls, axis=1)

        def rdma(src, dst, ssem, rsem, tgt):
            return pltpu.make_async_remote_copy(
                src_ref=src, dst_ref=dst, send_sem=ssem, recv_sem=rsem,
                device_id=(tgt,), device_id_type=pl.DeviceIdType.MESH)

        def q_rows(q):
            return pl.ds(q * QR, QR)

        UNIT_ROWS = {0: (0, 2 * QR), 1: (2 * QR, QR), 2: (3 * QR, QR)}

        def ag_sends(u, targets):
            row0, nrows = UNIT_ROWS[u]
            sl = pl.ds(row0, nrows)
            rs = [rdma(gctx.at[:, sl, :], gctx.at[:, sl, :],
                       ag_send.at[u, t], ag_recv.at[u], tgt)
                  for t, tgt in enumerate(targets)]
            for r in rs:
                r.start()
            return rs

        def ag_recv_wait(u):
            row0, nrows = UNIT_ROWS[u]
            sl = pl.ds(row0, nrows)
            rdma(gctx.at[:, sl, :], gctx.at[:, sl, :],
                 ag_send.at[u, 0], ag_recv.at[u], my).wait_recv()

        def merge_into_gctx(q, ctxA, statA, Arow0, ctxB, statB, Brow0):
            for b in range(B):
                blocks = []
                for h in range(HQ):
                    hs = slice(h * DH, (h + 1) * DH)
                    mA = statA[b, Arow0:Arow0 + QR, h:h + 1]
                    lA = statA[b, Arow0:Arow0 + QR, HQ + h:HQ + h + 1]
                    cA = ctxA[b, Arow0:Arow0 + QR, hs].astype(jnp.float32)
                    mB = statB[b, Brow0:Brow0 + QR, h:h + 1]
                    lB = statB[b, Brow0:Brow0 + QR, HQ + h:HQ + h + 1]
                    cB = ctxB[b, Brow0:Brow0 + QR, hs].astype(jnp.float32)
                    mg = jnp.maximum(mA, mB)
                    ea = jnp.exp(mA - mg)
                    eb = jnp.exp(mB - mg)
                    blocks.append(
                        (ea * cA + eb * cB) / (ea * lA + eb * lB))
                gctx[b, q * QR:(q + 1) * QR, :] = jnp.concatenate(
                    blocks, axis=1).astype(BF)

        @pl.when(my == 0)
        def _():
            compute_partial(2 * QR, SQ, 0, SKV_SHARD, False)
            sc = [
                rdma(loc_ctx.at[:, q_rows(2), :], rctx_a,
                     sc_send.at[0], sc_recv.at[0], right),
                rdma(loc_stat.at[:, q_rows(2), :], rstat_a,
                     st_send.at[0], st_recv.at[0], right),
                rdma(loc_ctx.at[:, q_rows(3), :], rctx_a,
                     sc_send.at[1], sc_recv.at[1], left),
                rdma(loc_stat.at[:, q_rows(3), :], rstat_a,
                     st_send.at[1], st_recv.at[1], left),
            ]
            for r in sc:
                r.start()
            compute_partial(0, 2 * QR, 0, SKV_SHARD, True)
            ag0 = ag_sends(0, [right, left, diag])
            ag_recv_wait(1)
            ag_recv_wait(2)
            for r in sc + ag0:
                r.wait_send()

        @pl.when(my == 1)
        def _():
            compute_partial(2 * QR, SQ, 0, 2 * QR, False)
            sc = [
                rdma(loc_ctx.at[:, q_rows(3), :], rctx_b,
                     sc_send.at[3], sc_recv.at[3], diag),
                rdma(loc_stat.at[:, q_rows(3), :], rstat_b,
                     st_send.at[2], st_recv.at[2], diag),
            ]
            for r in sc:
                r.start()
            rdma(rctx_a, rctx_a, sc_send.at[0], sc_recv.at[0], left).wait_recv()
            rdma(rstat_a, rstat_a, st_send.at[0], st_recv.at[0], left).wait_recv()
            merge_into_gctx(2, rctx_a, rstat_a, 0,
                            loc_ctx, loc_stat, 2 * QR)
            ag1 = ag_sends(1, [right, left, diag])
            ag_recv_wait(0)
            ag_recv_wait(2)
            for r in sc + ag1:
                r.wait_send()

        @pl.when(my == 2)
        def _():
            for u in range(3):
                ag_recv_wait(u)

        @pl.when(my == 3)
        def _():
            rdma(rctx_a, rctx_a, sc_send.at[1], sc_recv.at[1], right).wait_recv()
            rdma(rstat_a, rstat_a, st_send.at[1], st_recv.at[1], right).wait_recv()
            rdma(rctx_b, rctx_b, sc_send.at[3], sc_recv.at[3], diag).wait_recv()
            rdma(rstat_b, rstat_b, st_send.at[2], st_recv.at[2], diag).wait_recv()
            merge_into_gctx(3, rctx_a, rstat_a, 0, rctx_b, rstat_b, 0)
            ag2 = ag_sends(2, [right, left, diag])
            ag_recv_wait(0)
            ag_recv_wait(1)
            for r in ag2:
                r.wait_send()

        wo16 = wo_ref[:, :].astype(BF)
        for b in range(B):
            out_ref[b] = jnp.dot(
                gctx[b], wo16,
                preferred_element_type=jnp.float32).astype(BF)

    return pl.pallas_call(
        body,
        out_shape=jax.ShapeDtypeStruct((B, SQ, D_MODEL), BF),
        in_specs=[pl.BlockSpec(memory_space=pltpu.VMEM)] * 5,
        out_specs=pl.BlockSpec(memory_space=pltpu.VMEM),
        scratch_shapes=[
            pltpu.VMEM((B, SQ, HD), BF),
            pltpu.VMEM((B, SQ, 2 * HQ), jnp.float32),
            pltpu.VMEM((B, QR, HD), BF),
            pltpu.VMEM((B, QR, 2 * HQ), jnp.float32),
            pltpu.VMEM((B, QR, HD), BF),
            pltpu.VMEM((B, QR, 2 * HQ), jnp.float32),
            pltpu.VMEM((B, SQ, HD), BF),
            pltpu.SemaphoreType.DMA((4,)),
            pltpu.SemaphoreType.DMA((4,)),
            pltpu.SemaphoreType.DMA((3,)),
            pltpu.SemaphoreType.DMA((3,)),
            pltpu.SemaphoreType.DMA((3, 3)),
            pltpu.SemaphoreType.DMA((3,)),
        ],
        compiler_params=pltpu.CompilerParams(collective_id=0),
    )(x, Wq, K2, V2, Wo)
